# baseline (speedup 1.0000x reference)
import jax
import jax.numpy as jnp
from jax.experimental import pallas as pl
from jax.experimental.pallas import tpu as pltpu

VOCAB_SIZE = 25107
EMB_DIM = 100
CTX_LEN = 5
BATCH = 16
H1 = 300
H2 = 300

K_CHUNK = 64
NUM_FULL_CHUNKS = 4
K_TAIL = H2 - NUM_FULL_CHUNKS * K_CHUNK


def _mlp_kernel(x_smem, emb_hbm, w1_ref, b1_ref, w2_ref, b2_ref, w3_hbm,
                b3_ref, out_ref, ebuf, bufs, tail_buf, gsem, sems, tail_sem):
    copies = []
    for j in range(NUM_FULL_CHUNKS):
        c = pltpu.make_async_copy(
            w3_hbm.at[pl.ds(j * K_CHUNK, K_CHUNK), :], bufs.at[j],
            sems.at[j])
        c.start()
        copies.append(c)
    tail_copy = pltpu.make_async_copy(
        w3_hbm.at[pl.ds(NUM_FULL_CHUNKS * K_CHUNK, K_TAIL), :], tail_buf,
        tail_sem)
    tail_copy.start()

    gathers = []
    for b in range(BATCH):
        for c in range(CTX_LEN):
            g = pltpu.make_async_copy(
                emb_hbm.at[pl.ds(x_smem[b, c], 1), :],
                ebuf.at[c, pl.ds(b, 1), :], gsem)
            g.start()
            gathers.append(g)
    for g in gathers:
        g.wait()

    h1 = b1_ref[...][None, :]
    for c in range(CTX_LEN):
        h1 = h1 + jnp.dot(ebuf[c], w1_ref[c],
                          preferred_element_type=jnp.float32)
    h1 = jnp.maximum(h1, 0.0)
    h2 = jnp.maximum(
        jnp.dot(h1, w2_ref[...],
                preferred_element_type=jnp.float32) + b2_ref[...][None, :],
        0.0)

    for j in range(NUM_FULL_CHUNKS):
        copies[j].wait()
        part = jnp.dot(h2[:, j * K_CHUNK:(j + 1) * K_CHUNK], bufs[j],
                       preferred_element_type=jnp.float32)
        if j == 0:
            out_ref[...] = part + b3_ref[...][None, :]
        else:
            out_ref[...] = out_ref[...] + part

    tail_copy.wait()
    part = jnp.dot(h2[:, NUM_FULL_CHUNKS * K_CHUNK:], tail_buf[...],
                   preferred_element_type=jnp.float32)
    out_ref[...] = out_ref[...] + part


def kernel(x, emb, W1, b1, W2, b2, W3, b3):
    return pl.pallas_call(
        _mlp_kernel,
        in_specs=[
            pl.BlockSpec(memory_space=pltpu.SMEM),
            pl.BlockSpec(memory_space=pl.ANY),
            pl.BlockSpec(memory_space=pltpu.VMEM),
            pl.BlockSpec(memory_space=pltpu.VMEM),
            pl.BlockSpec(memory_space=pltpu.VMEM),
            pl.BlockSpec(memory_space=pltpu.VMEM),
            pl.BlockSpec(memory_space=pl.ANY),
            pl.BlockSpec(memory_space=pltpu.VMEM),
        ],
        out_specs=pl.BlockSpec(memory_space=pltpu.VMEM),
        out_shape=jax.ShapeDtypeStruct((BATCH, VOCAB_SIZE), jnp.float32),
        scratch_shapes=[
            pltpu.VMEM((CTX_LEN, BATCH, EMB_DIM), jnp.float32),
            pltpu.VMEM((NUM_FULL_CHUNKS, K_CHUNK, VOCAB_SIZE), jnp.float32),
            pltpu.VMEM((K_TAIL, VOCAB_SIZE), jnp.float32),
            pltpu.SemaphoreType.DMA,
            pltpu.SemaphoreType.DMA((NUM_FULL_CHUNKS,)),
            pltpu.SemaphoreType.DMA,
        ],
    )(x, emb, W1.reshape(CTX_LEN, EMB_DIM, H1), b1, W2, b2, W3, b3)
